# Initial kernel scaffold; baseline (speedup 1.0000x reference)
import jax
import jax.numpy as jnp
from jax.experimental import pallas as pl


def kernel(logprobs):
    raise NotImplementedError("write your pallas kernel here")


# speedup vs baseline: 152.2912x; 152.2912x over previous
import functools

import jax
import jax.numpy as jnp
from jax.experimental import pallas as pl

TOP_P = 0.9
NEG_INF = float("-inf")
N_BISECT = 26

ROWS = 32
SUB = 7816
LANE = 128
PADDED = SUB * LANE


def _row_kernel(x_ref, o_ref):
    x = x_ref[0]
    m = jnp.max(x)
    p = jnp.exp(x - m)
    z = jnp.sum(p)
    target = TOP_P * z

    def body(_, carry):
        lo, hi, s_lo = carry
        mid = 0.5 * (lo + hi)
        s = jnp.sum(jnp.where(p > mid, p, 0.0))
        ge = s >= target
        lo2 = jnp.where(ge, mid, lo)
        hi2 = jnp.where(ge, hi, mid)
        s2 = jnp.where(ge, s, s_lo)
        return (lo2, hi2, s2)

    lo, _, s_lo = jax.lax.fori_loop(0, N_BISECT, body, (0.0, 1.0, z))
    c = m + jnp.log(s_lo)
    o_ref[0] = jnp.where(p > lo, x - c, NEG_INF)


@jax.jit
def kernel(logprobs):
    b, v = logprobs.shape
    x = jnp.pad(logprobs, ((0, 0), (0, PADDED - v)), constant_values=NEG_INF)
    x = x.reshape(b, SUB, LANE)
    out = pl.pallas_call(
        _row_kernel,
        grid=(b,),
        in_specs=[pl.BlockSpec((1, SUB, LANE), lambda i: (i, 0, 0))],
        out_specs=pl.BlockSpec((1, SUB, LANE), lambda i: (i, 0, 0)),
        out_shape=jax.ShapeDtypeStruct((b, SUB, LANE), jnp.float32),
    )(x)
    return out.reshape(b, PADDED)[:, :v]
